# baseline (speedup 1.0000x reference)
import functools

import jax
import jax.numpy as jnp
from jax.experimental import pallas as pl
from jax.experimental.pallas import tpu as pltpu

_B = 16


def _body(x_ref, seg_ref, w1_ref, b1_ref, w2_ref, out_ref,
          d_ref, acc_ref, *, nblk, blk_r):
    j = pl.program_id(0)

    @pl.when(j == 0)
    def _init():
        d_ref[...] = jnp.zeros_like(d_ref)
        acc_ref[...] = jnp.zeros_like(acc_ref)

    xb = x_ref[...]
    h = jnp.tanh(jnp.dot(xb, w1_ref[...],
                         preferred_element_type=jnp.float32) + b1_ref[...])

    w2 = w2_ref[...]
    w2t = jnp.broadcast_to(w2, (w2.shape[0], _B))
    shift = -jnp.sum(jnp.abs(w2))
    s = jnp.dot(h, w2t, preferred_element_type=jnp.float32) + shift

    ids = seg_ref[0]
    seg_iota = jax.lax.broadcasted_iota(jnp.int32, (blk_r, _B), 1)
    onehot = ids.reshape(blk_r, 1) == seg_iota

    eB = jnp.where(onehot, jnp.exp(s), 0.0)
    d_ref[...] += jnp.sum(eB, axis=0, keepdims=True)
    part = jax.lax.dot_general(eB, xb, (((0,), (0,)), ((), ())),
                               preferred_element_type=jnp.float32)
    acc_ref[...] += part

    @pl.when(j == nblk - 1)
    def _finish():
        d = d_ref[...].reshape(_B, 1)
        out_ref[...] = acc_ref[...] / jnp.where(d == 0.0, 1.0, d)


@jax.jit
def kernel(x, batch, W1, b1, W2, b2):
    N, D = x.shape
    H = W1.shape[1]
    blk_r = 2048
    nblk = N // blk_r

    seg = batch.astype(jnp.int32).reshape(nblk, 1, blk_r)
    b1r = b1.reshape(1, H)

    grid_spec = pltpu.PrefetchScalarGridSpec(
        num_scalar_prefetch=0,
        grid=(nblk,),
        in_specs=[
            pl.BlockSpec((blk_r, D), lambda j: (j, 0)),
            pl.BlockSpec((1, 1, blk_r), lambda j: (j, 0, 0)),
            pl.BlockSpec((D, H), lambda j: (0, 0)),
            pl.BlockSpec((1, H), lambda j: (0, 0)),
            pl.BlockSpec((H, 1), lambda j: (0, 0)),
        ],
        out_specs=pl.BlockSpec((_B, D), lambda j: (0, 0)),
        scratch_shapes=[
            pltpu.VMEM((1, _B), jnp.float32),
            pltpu.VMEM((_B, D), jnp.float32),
        ],
    )

    out = pl.pallas_call(
        functools.partial(_body, nblk=nblk, blk_r=blk_r),
        grid_spec=grid_spec,
        out_shape=jax.ShapeDtypeStruct((_B, D), jnp.float32),
        compiler_params=pltpu.CompilerParams(
            dimension_semantics=("arbitrary",),
        ),
    )(x, seg, W1, b1r, W2)
    return out


# speedup vs baseline: 1.1416x; 1.1416x over previous
import functools

import jax
import jax.numpy as jnp
from jax.experimental import pallas as pl
from jax.experimental.pallas import tpu as pltpu

_B = 16


def _body(x_ref, seg_ref, w1_ref, b1_ref, w2_ref, out_ref,
          d_ref, acc_ref, *, nblk, blk_r):
    j = pl.program_id(0)

    @pl.when(j == 0)
    def _init():
        d_ref[...] = jnp.zeros_like(d_ref)
        acc_ref[...] = jnp.zeros_like(acc_ref)

    w2 = w2_ref[...]
    w2t = jnp.broadcast_to(w2, (w2.shape[0], _B))
    shift = -jnp.sum(jnp.abs(w2))
    ids = seg_ref[0]

    hr = blk_r // 2
    seg_iota = jax.lax.broadcasted_iota(jnp.int32, (hr, _B), 1)

    def _half(lo):
        xh = x_ref[pl.ds(lo, hr), :]
        hh = jnp.tanh(jnp.dot(xh, w1_ref[...],
                              preferred_element_type=jnp.float32)
                      + b1_ref[...])
        sh = jnp.dot(hh, w2t,
                     preferred_element_type=jnp.float32) + shift
        oh = ids[:, lo:lo + hr].reshape(hr, 1) == seg_iota
        e = jnp.where(oh, jnp.exp(sh), 0.0)
        p = jax.lax.dot_general(e, xh, (((0,), (0,)), ((), ())),
                                preferred_element_type=jnp.float32)
        return jnp.sum(e, axis=0, keepdims=True), p

    da, pa = _half(0)
    db, pb = _half(hr)
    d_ref[...] += da + db
    acc_ref[...] += pa + pb

    @pl.when(j == nblk - 1)
    def _finish():
        d = d_ref[...].reshape(_B, 1)
        out_ref[...] = acc_ref[...] / jnp.where(d == 0.0, 1.0, d)


@jax.jit
def kernel(x, batch, W1, b1, W2, b2):
    N, D = x.shape
    H = W1.shape[1]
    blk_r = 2048
    nblk = N // blk_r

    seg = batch.astype(jnp.int32).reshape(nblk, 1, blk_r)
    b1r = b1.reshape(1, H)

    grid_spec = pltpu.PrefetchScalarGridSpec(
        num_scalar_prefetch=0,
        grid=(nblk,),
        in_specs=[
            pl.BlockSpec((blk_r, D), lambda j: (j, 0)),
            pl.BlockSpec((1, 1, blk_r), lambda j: (j, 0, 0)),
            pl.BlockSpec((D, H), lambda j: (0, 0)),
            pl.BlockSpec((1, H), lambda j: (0, 0)),
            pl.BlockSpec((H, 1), lambda j: (0, 0)),
        ],
        out_specs=pl.BlockSpec((_B, D), lambda j: (0, 0)),
        scratch_shapes=[
            pltpu.VMEM((1, _B), jnp.float32),
            pltpu.VMEM((_B, D), jnp.float32),
        ],
    )

    out = pl.pallas_call(
        functools.partial(_body, nblk=nblk, blk_r=blk_r),
        grid_spec=grid_spec,
        out_shape=jax.ShapeDtypeStruct((_B, D), jnp.float32),
        compiler_params=pltpu.CompilerParams(
            dimension_semantics=("arbitrary",),
        ),
    )(x, seg, W1, b1r, W2)
    return out
